# baseline (speedup 1.0000x reference)
import functools

import jax
import jax.numpy as jnp
from jax import lax
from jax.experimental import pallas as pl
from jax.experimental.pallas import tpu as pltpu
from jax.experimental.pallas import tpu_sc as plsc

L = 16
CHUNK = 128
NB = 5
TC_BLK = 4000
G = 8

_info = plsc.get_sparse_core_info()
NW = _info.num_cores * _info.num_subcores
NRG = NW // G
_MESH = dict(core_axis_name="c", subcore_axis_name="s")


def _worker_id():
    return lax.axis_index("s") * _info.num_cores + lax.axis_index("c")


def _sc_scan(idx, n_rows):
    B = idx.shape[0]
    N = n_rows
    SR = -(-N // NW // 8) * 8
    RG = NRG * 0 + (NW // NRG) * SR
    RG = (NW // NRG) * SR
    BS = B // G
    assert B % G == 0 and BS % L == 0 and RG % L == 0

    @functools.partial(
        pl.kernel,
        out_type=jax.ShapeDtypeStruct((G * NRG * RG,), jnp.int32),
        mesh=plsc.VectorSubcoreMesh(**_MESH),
        compiler_params=pltpu.CompilerParams(needs_layout_passes=False),
        scratch_types=[
            pltpu.VMEM((BS,), jnp.int32),
            pltpu.VMEM((RG,), jnp.int32),
        ],
    )
    def k(idx_h, tmp_h, idx_v, tmp):
        wid = _worker_id()
        g = wid // NRG
        r = wid % NRG
        lo = r * RG
        hi = jnp.minimum(lo + RG, N)

        pltpu.sync_copy(idx_h.at[pl.ds(g * BS, BS)], idx_v)

        neg1 = jnp.full((L,), -1, jnp.int32)

        def init_body(i, _):
            for u in range(4):
                tmp[pl.ds((i * 4 + u) * L, L)] = neg1
            return 0
        lax.fori_loop(0, RG // L // 4, init_body, 0)

        iota = lax.iota(jnp.int32, L)
        pos0 = g * BS

        def p1(v, _):
            rows = idx_v[pl.ds(v * L, L)]
            m = (rows >= lo) & (rows < hi)
            local = jnp.where(m, rows - lo, 0)
            pos = iota + (pos0 + v * L)
            _, last_m = plsc.scan_count(local, mask=m)
            plsc.store_scatter(tmp, [local], pos, mask=last_m & m)
            return 0
        lax.fori_loop(0, BS // L, p1, 0)

        pltpu.sync_copy(tmp, tmp_h.at[pl.ds((g * NRG + r) * RG, RG)])

    return k(idx)


def _tc_copy(cell, hidden):
    N, D = cell.shape
    nb = N // TC_BLK

    def body(c_ref, h_ref, o_ref):
        o_ref[0] = c_ref[...]
        o_ref[1] = h_ref[...]

    return pl.pallas_call(
        body,
        grid=(nb,),
        in_specs=[
            pl.BlockSpec((TC_BLK, D), lambda i: (i, 0)),
            pl.BlockSpec((TC_BLK, D), lambda i: (i, 0)),
        ],
        out_specs=pl.BlockSpec((2, TC_BLK, D), lambda i: (0, i, 0)),
        out_shape=jax.ShapeDtypeStruct((2, N, D), jnp.float32),
    )(cell, hidden)


def _sc_scatter(out_ref, tmp_all, values_cell, values_hidden, n_rows):
    N = n_rows
    D = values_cell.shape[1]
    RG = tmp_all.shape[0] // (G * NRG)
    SR = RG // (NW // NRG)
    srpad = ((SR + L - 1) // L) * L
    wcap = (srpad // CHUNK + 2) * CHUNK

    @functools.partial(
        pl.kernel,
        mesh=plsc.VectorSubcoreMesh(**_MESH),
        compiler_params=pltpu.CompilerParams(needs_layout_passes=False),
        scratch_types=[pltpu.VMEM((srpad,), jnp.int32)] * G + [
            pltpu.VMEM((wcap,), jnp.int32),
            pltpu.VMEM((wcap,), jnp.int32),
            pltpu.VMEM((NB, CHUNK), jnp.int32),
            pltpu.VMEM((NB, CHUNK), jnp.int32),
            pltpu.VMEM((NB, CHUNK, D), jnp.float32),
        ] + [pltpu.SemaphoreType.DMA] * (2 * NB),
    )
    def k(tmp_h, vc_h, vh_h, out_h, *rest):
        slices = rest[:G]
        (win_row, win_pos, dst2d, src2d, stage) = rest[G:G + 5]
        sems = rest[G + 5:]
        sems_g = sems[:NB]
        sems_s = sems[NB:]
        wid = _worker_id()
        lo = wid * SR
        r = wid // (NW // NRG)
        off = (wid % (NW // NRG)) * SR

        cps = [
            pltpu.make_async_copy(
                tmp_h.at[pl.ds((g * NRG + r) * RG + off, SR)],
                slices[g].at[pl.ds(0, SR)], sems_g[0])
            for g in range(G)
        ]
        for cp in cps:
            cp.start()
        for cp in cps:
            cp.wait()

        iota = lax.iota(jnp.int32, L)

        def fire_cell(b, ci):
            def ld(j, _):
                d2 = dst2d.at[b]
                s2 = src2d.at[b]
                d2[pl.ds(j * L, L)] = win_row[pl.ds(ci * CHUNK + j * L, L)]
                s2[pl.ds(j * L, L)] = win_pos[pl.ds(ci * CHUNK + j * L, L)]
                return 0
            lax.fori_loop(0, CHUNK // L, ld, 0)
            pltpu.make_async_copy(
                vc_h.at[src2d.at[b]], stage.at[b], sems_g[b]).start()

        def p2(t, carry):
            cnt, nf = carry
            w = slices[0][pl.ds(t * L, L)]
            for g in range(1, G):
                tg = slices[g][pl.ds(t * L, L)]
                w = jnp.where(tg >= 0, tg, w)
            m = (w >= 0) & (iota + t * L < SR)
            rows16 = iota + t * L + lo
            plsc.store_compressed(win_row.at[pl.ds(cnt, L)], rows16, mask=m)
            plsc.store_compressed(win_pos.at[pl.ds(cnt, L)], w, mask=m)
            cnt2 = cnt + plsc.all_reduce_population_count(m)[0]
            for b in range(NB):
                @pl.when((nf == b) & (cnt2 >= (b + 1) * CHUNK))
                def _(b=b):
                    fire_cell(b, jnp.int32(b))
            nf2 = nf + jnp.where(
                (nf < NB) & (cnt2 >= (nf + 1) * CHUNK), 1, 0).astype(jnp.int32)
            return (cnt2, nf2)
        cnt, nfired = lax.fori_loop(
            0, srpad // L, p2, (jnp.int32(0), jnp.int32(0)))

        @pl.when(cnt > 0)
        def _pad():
            frv = jnp.full((L,), win_row[pl.ds(0, L)][0], jnp.int32)
            fpv = jnp.full((L,), win_pos[pl.ds(0, L)][0], jnp.int32)
            for j in range(CHUNK // L):
                win_row[pl.ds(cnt + j * L, L)] = frv
                win_pos[pl.ds(cnt + j * L, L)] = fpv

        nch_w = (cnt + CHUNK - 1) // CHUNK
        ntot = 2 * nch_w

        def fire_gather(b, c):
            t_is_h = c >= nch_w
            ci = jnp.where(t_is_h, c - nch_w, c)
            base = jnp.where(t_is_h, N, 0)

            def ld(j, _):
                d2 = dst2d.at[b]
                s2 = src2d.at[b]
                d2[pl.ds(j * L, L)] = (
                    win_row[pl.ds(ci * CHUNK + j * L, L)] + base)
                s2[pl.ds(j * L, L)] = win_pos[pl.ds(ci * CHUNK + j * L, L)]
                return 0
            lax.fori_loop(0, CHUNK // L, ld, 0)

            @pl.when(jnp.logical_not(t_is_h))
            def _():
                pltpu.make_async_copy(
                    vc_h.at[src2d.at[b]], stage.at[b], sems_g[b]).start()

            @pl.when(t_is_h)
            def _():
                pltpu.make_async_copy(
                    vh_h.at[src2d.at[b]], stage.at[b], sems_g[b]).start()

        def drain(sem, b):
            pltpu.make_async_copy(
                vc_h.at[pl.ds(0, CHUNK)], stage.at[b], sem).wait()

        for b in range(NB):
            @pl.when((b >= nfired) & (b < ntot))
            def _(b=b):
                fire_gather(b, jnp.int32(b))

        def ring(g2, _):
            for b in range(NB):
                c = g2 * NB + b

                @pl.when(c < ntot)
                def _(b=b, c=c):
                    drain(sems_g[b], b)
                    pltpu.make_async_copy(
                        stage.at[b], out_h.at[dst2d.at[b]], sems_s[b]).start()
            for b in range(NB):
                c2 = (g2 + 1) * NB + b

                @pl.when(c2 < ntot)
                def _(b=b, c2=c2):
                    drain(sems_s[b], b)
                    fire_gather(b, c2)
            return 0
        ngroups = (ntot + NB - 1) // NB
        lax.fori_loop(0, ngroups, ring, 0)
        for b in range(NB):
            @pl.when(b < ntot)
            def _(b=b):
                drain(sems_s[b], b)

    k(tmp_all, values_cell, values_hidden, out_ref)


def kernel(cell, hidden, node_idxs, values_cell, values_hidden):
    N, D = cell.shape
    idx = node_idxs.astype(jnp.int32)
    out0 = _tc_copy(cell, hidden)
    tmp_all = _sc_scan(idx, N)
    out_ref = jax.new_ref(out0.reshape(2 * N, D))
    _sc_scatter(out_ref, tmp_all, values_cell, values_hidden, N)
    return out_ref[...].reshape(2, N, D)


# speedup vs baseline: 1.0257x; 1.0257x over previous
import functools

import jax
import jax.numpy as jnp
from jax import lax
from jax.experimental import pallas as pl
from jax.experimental.pallas import tpu as pltpu
from jax.experimental.pallas import tpu_sc as plsc

L = 16
CHUNK = 128
NB = 5
TC_BLK = 4000
G = 8

_info = plsc.get_sparse_core_info()
NW = _info.num_cores * _info.num_subcores
NRG = NW // G
_MESH = dict(core_axis_name="c", subcore_axis_name="s")


def _worker_id():
    return lax.axis_index("s") * _info.num_cores + lax.axis_index("c")


def _sc_scan(idx, n_rows):
    B = idx.shape[0]
    N = n_rows
    SR = -(-N // NW // 8) * 8
    RG = NRG * 0 + (NW // NRG) * SR
    RG = (NW // NRG) * SR
    BS = B // G
    assert B % G == 0 and BS % L == 0 and RG % L == 0

    @functools.partial(
        pl.kernel,
        out_type=jax.ShapeDtypeStruct((G * NRG * RG,), jnp.int32),
        mesh=plsc.VectorSubcoreMesh(**_MESH),
        compiler_params=pltpu.CompilerParams(needs_layout_passes=False),
        scratch_types=[
            pltpu.VMEM((BS,), jnp.int32),
            pltpu.VMEM((RG,), jnp.int32),
        ],
    )
    def k(idx_h, tmp_h, idx_v, tmp):
        wid = _worker_id()
        g = wid // NRG
        r = wid % NRG
        lo = r * RG
        hi = jnp.minimum(lo + RG, N)

        pltpu.sync_copy(idx_h.at[pl.ds(g * BS, BS)], idx_v)

        neg1 = jnp.full((L,), -1, jnp.int32)

        def init_body(i, _):
            for u in range(4):
                tmp[pl.ds((i * 4 + u) * L, L)] = neg1
            return 0
        lax.fori_loop(0, RG // L // 4, init_body, 0)

        iota = lax.iota(jnp.int32, L)
        pos0 = g * BS

        def p1(v, _):
            rows = idx_v[pl.ds(v * L, L)]
            m = (rows >= lo) & (rows < hi)
            local = jnp.where(m, rows - lo, 0)
            pos = iota + (pos0 + v * L)
            _, last_m = plsc.scan_count(local, mask=m)
            plsc.store_scatter(tmp, [local], pos, mask=last_m & m)
            return 0
        lax.fori_loop(0, BS // L, p1, 0)

        pltpu.sync_copy(tmp, tmp_h.at[pl.ds((g * NRG + r) * RG, RG)])

    return k(idx)


def _tc_copy(cell, hidden):
    N, D = cell.shape
    nb = N // TC_BLK

    def body(c_ref, h_ref, o_ref):
        o_ref[0] = c_ref[...]
        o_ref[1] = h_ref[...]

    return pl.pallas_call(
        body,
        grid=(nb,),
        in_specs=[
            pl.BlockSpec((TC_BLK, D), lambda i: (i, 0)),
            pl.BlockSpec((TC_BLK, D), lambda i: (i, 0)),
        ],
        out_specs=pl.BlockSpec((2, TC_BLK, D), lambda i: (0, i, 0)),
        out_shape=jax.ShapeDtypeStruct((2, N, D), jnp.float32),
    )(cell, hidden)


def _sc_scatter(out_ref, tmp_all, values_cell, values_hidden, n_rows):
    N = n_rows
    D = values_cell.shape[1]
    RG = tmp_all.shape[0] // (G * NRG)
    SR = RG // (NW // NRG)
    srpad = ((SR + L - 1) // L) * L
    wcap = (srpad // CHUNK + 2) * CHUNK

    @functools.partial(
        pl.kernel,
        mesh=plsc.VectorSubcoreMesh(**_MESH),
        compiler_params=pltpu.CompilerParams(needs_layout_passes=False),
        scratch_types=[pltpu.VMEM((srpad,), jnp.int32)] * G + [
            pltpu.VMEM((wcap,), jnp.int32),
            pltpu.VMEM((wcap,), jnp.int32),
            pltpu.VMEM((NB, CHUNK), jnp.int32),
            pltpu.VMEM((NB, CHUNK), jnp.int32),
            pltpu.VMEM((NB, CHUNK, D), jnp.float32),
        ] + [pltpu.SemaphoreType.DMA] * (2 * NB),
    )
    def k(tmp_h, vc_h, vh_h, out_h, *rest):
        slices = rest[:G]
        (win_row, win_pos, dst2d, src2d, stage) = rest[G:G + 5]
        sems = rest[G + 5:]
        sems_g = sems[:NB]
        sems_s = sems[NB:]
        wid = _worker_id()
        lo = wid * SR
        r = wid // (NW // NRG)
        off = (wid % (NW // NRG)) * SR

        cps = [
            pltpu.make_async_copy(
                tmp_h.at[pl.ds((g * NRG + r) * RG + off, SR)],
                slices[g].at[pl.ds(0, SR)], sems_g[0])
            for g in range(G)
        ]
        for cp in cps:
            cp.start()
        for cp in cps:
            cp.wait()

        iota = lax.iota(jnp.int32, L)

        def fire_cell0():
            def ld(j, _):
                dst2d.at[0][pl.ds(j * L, L)] = win_row[pl.ds(j * L, L)]
                src2d.at[0][pl.ds(j * L, L)] = win_pos[pl.ds(j * L, L)]
                return 0
            lax.fori_loop(0, CHUNK // L, ld, 0)
            pltpu.make_async_copy(
                vc_h.at[src2d.at[0]], stage.at[0], sems_g[0]).start()

        def p2(t, carry):
            cnt, fired = carry
            w = slices[0][pl.ds(t * L, L)]
            for g in range(1, G):
                tg = slices[g][pl.ds(t * L, L)]
                w = jnp.where(tg >= 0, tg, w)
            m = (w >= 0) & (iota + t * L < SR)
            rows16 = iota + t * L + lo
            plsc.store_compressed(win_row.at[pl.ds(cnt, L)], rows16, mask=m)
            plsc.store_compressed(win_pos.at[pl.ds(cnt, L)], w, mask=m)
            cnt2 = cnt + plsc.all_reduce_population_count(m)[0]
            hit = jnp.logical_not(fired) & (cnt2 >= CHUNK)

            @pl.when(hit)
            def _():
                fire_cell0()
            return (cnt2, fired | hit)
        cnt, fired0 = lax.fori_loop(
            0, srpad // L, p2, (jnp.int32(0), jnp.bool_(False)))

        @pl.when(cnt > 0)
        def _pad():
            frv = jnp.full((L,), win_row[pl.ds(0, L)][0], jnp.int32)
            fpv = jnp.full((L,), win_pos[pl.ds(0, L)][0], jnp.int32)
            for j in range(CHUNK // L):
                win_row[pl.ds(cnt + j * L, L)] = frv
                win_pos[pl.ds(cnt + j * L, L)] = fpv

        nch_w = (cnt + CHUNK - 1) // CHUNK
        ntot = 2 * nch_w

        def fire_gather(b, c):
            t_is_h = c >= nch_w
            ci = jnp.where(t_is_h, c - nch_w, c)
            base = jnp.where(t_is_h, N, 0)

            def ld(j, _):
                d2 = dst2d.at[b]
                s2 = src2d.at[b]
                d2[pl.ds(j * L, L)] = (
                    win_row[pl.ds(ci * CHUNK + j * L, L)] + base)
                s2[pl.ds(j * L, L)] = win_pos[pl.ds(ci * CHUNK + j * L, L)]
                return 0
            lax.fori_loop(0, CHUNK // L, ld, 0)

            @pl.when(jnp.logical_not(t_is_h))
            def _():
                pltpu.make_async_copy(
                    vc_h.at[src2d.at[b]], stage.at[b], sems_g[b]).start()

            @pl.when(t_is_h)
            def _():
                pltpu.make_async_copy(
                    vh_h.at[src2d.at[b]], stage.at[b], sems_g[b]).start()

        def drain(sem, b):
            pltpu.make_async_copy(
                vc_h.at[pl.ds(0, CHUNK)], stage.at[b], sem).wait()

        @pl.when(jnp.logical_not(fired0) & (0 < ntot))
        def _():
            fire_gather(0, jnp.int32(0))
        for b in range(1, NB):
            @pl.when(b < ntot)
            def _(b=b):
                fire_gather(b, jnp.int32(b))

        def ring(g2, _):
            for b in range(NB):
                c = g2 * NB + b

                @pl.when(c < ntot)
                def _(b=b, c=c):
                    drain(sems_g[b], b)
                    pltpu.make_async_copy(
                        stage.at[b], out_h.at[dst2d.at[b]], sems_s[b]).start()
            for b in range(NB):
                c2 = (g2 + 1) * NB + b

                @pl.when(c2 < ntot)
                def _(b=b, c2=c2):
                    drain(sems_s[b], b)
                    fire_gather(b, c2)
            return 0
        ngroups = (ntot + NB - 1) // NB
        lax.fori_loop(0, ngroups, ring, 0)
        for b in range(NB):
            @pl.when(b < ntot)
            def _(b=b):
                drain(sems_s[b], b)

    k(tmp_all, values_cell, values_hidden, out_ref)


def kernel(cell, hidden, node_idxs, values_cell, values_hidden):
    N, D = cell.shape
    idx = node_idxs.astype(jnp.int32)
    out0 = _tc_copy(cell, hidden)
    tmp_all = _sc_scan(idx, N)
    out_ref = jax.new_ref(out0.reshape(2 * N, D))
    _sc_scatter(out_ref, tmp_all, values_cell, values_hidden, N)
    return out_ref[...].reshape(2, N, D)


# speedup vs baseline: 1.0571x; 1.0306x over previous
import functools

import jax
import jax.numpy as jnp
from jax import lax
from jax.experimental import pallas as pl
from jax.experimental.pallas import tpu as pltpu
from jax.experimental.pallas import tpu_sc as plsc

L = 16
CHUNK = 128
NB = 5
TC_BLK = 4000
G = 8

_info = plsc.get_sparse_core_info()
NW = _info.num_cores * _info.num_subcores
NRG = NW // G
_MESH = dict(core_axis_name="c", subcore_axis_name="s")


def _worker_id():
    return lax.axis_index("s") * _info.num_cores + lax.axis_index("c")


def _sc_scan(idx, n_rows):
    B = idx.shape[0]
    N = n_rows
    SR = -(-N // NW // 8) * 8
    RG = NRG * 0 + (NW // NRG) * SR
    RG = (NW // NRG) * SR
    BS = B // G
    assert B % G == 0 and BS % L == 0 and RG % L == 0

    @functools.partial(
        pl.kernel,
        out_type=jax.ShapeDtypeStruct((G * NRG * RG,), jnp.int32),
        mesh=plsc.VectorSubcoreMesh(**_MESH),
        compiler_params=pltpu.CompilerParams(needs_layout_passes=False),
        scratch_types=[
            pltpu.VMEM((BS,), jnp.int32),
            pltpu.VMEM((RG,), jnp.int32),
        ],
    )
    def k(idx_h, tmp_h, idx_v, tmp):
        wid = _worker_id()
        g = wid // NRG
        r = wid % NRG
        lo = r * RG
        hi = jnp.minimum(lo + RG, N)

        pltpu.sync_copy(idx_h.at[pl.ds(g * BS, BS)], idx_v)

        neg1 = jnp.full((L,), -1, jnp.int32)

        def init_body(i, _):
            for u in range(4):
                tmp[pl.ds((i * 4 + u) * L, L)] = neg1
            return 0
        lax.fori_loop(0, RG // L // 4, init_body, 0)

        iota = lax.iota(jnp.int32, L)
        pos0 = g * BS

        def p1(v, _):
            rows = idx_v[pl.ds(v * L, L)]
            m = (rows >= lo) & (rows < hi)
            local = jnp.where(m, rows - lo, 0)
            pos = iota + (pos0 + v * L)
            _, last_m = plsc.scan_count(local, mask=m)
            plsc.store_scatter(tmp, [local], pos, mask=last_m & m)
            return 0
        lax.fori_loop(0, BS // L, p1, 0)

        pltpu.sync_copy(tmp, tmp_h.at[pl.ds((g * NRG + r) * RG, RG)])

    return k(idx)


def _tc_copy(cell, hidden):
    N, D = cell.shape
    nb = N // TC_BLK

    def body(c_ref, h_ref, o_ref):
        o_ref[0] = c_ref[...]
        o_ref[1] = h_ref[...]

    return pl.pallas_call(
        body,
        grid=(nb,),
        in_specs=[
            pl.BlockSpec((TC_BLK, D), lambda i: (i, 0)),
            pl.BlockSpec((TC_BLK, D), lambda i: (i, 0)),
        ],
        out_specs=pl.BlockSpec((2, TC_BLK, D), lambda i: (0, i, 0)),
        out_shape=jax.ShapeDtypeStruct((2, N, D), jnp.float32),
    )(cell, hidden)


def _sc_scatter(out_ref, tmp_all, values_cell, values_hidden, n_rows):
    N = n_rows
    D = values_cell.shape[1]
    RG = tmp_all.shape[0] // (G * NRG)
    SR = RG // (NW // NRG)
    srpad = ((SR + L - 1) // L) * L
    wcap = (srpad // CHUNK + 2) * CHUNK

    @functools.partial(
        pl.kernel,
        mesh=plsc.VectorSubcoreMesh(**_MESH),
        compiler_params=pltpu.CompilerParams(needs_layout_passes=False),
        scratch_types=[pltpu.VMEM((srpad,), jnp.int32)] * G + [
            pltpu.VMEM((wcap,), jnp.int32),
            pltpu.VMEM((wcap,), jnp.int32),
            pltpu.VMEM((NB, CHUNK), jnp.int32),
            pltpu.VMEM((NB, CHUNK), jnp.int32),
            pltpu.VMEM((NB, CHUNK, D), jnp.float32),
        ] + [pltpu.SemaphoreType.DMA] * (2 * NB),
    )
    def k(tmp_h, vc_h, vh_h, out_h, *rest):
        slices = rest[:G]
        (win_row, win_pos, dst2d, src2d, stage) = rest[G:G + 5]
        sems = rest[G + 5:]
        sems_g = sems[:NB]
        sems_s = sems[NB:]
        wid = _worker_id()
        lo = wid * SR
        r = wid // (NW // NRG)
        off = (wid % (NW // NRG)) * SR

        cps = [
            pltpu.make_async_copy(
                tmp_h.at[pl.ds((g * NRG + r) * RG + off, SR)],
                slices[g].at[pl.ds(0, SR)], sems_g[0])
            for g in range(G)
        ]
        for cp in cps:
            cp.start()
        for cp in cps:
            cp.wait()

        iota = lax.iota(jnp.int32, L)

        def p2(t, cnt):
            w = slices[0][pl.ds(t * L, L)]
            for g in range(1, G):
                tg = slices[g][pl.ds(t * L, L)]
                w = jnp.where(tg >= 0, tg, w)
            m = (w >= 0) & (iota + t * L < SR)
            rows16 = iota + t * L + lo
            plsc.store_compressed(win_row.at[pl.ds(cnt, L)], rows16, mask=m)
            plsc.store_compressed(win_pos.at[pl.ds(cnt, L)], w, mask=m)
            return cnt + plsc.all_reduce_population_count(m)[0]
        cnt = lax.fori_loop(0, srpad // L, p2, jnp.int32(0))

        @pl.when(cnt > 0)
        def _pad():
            frv = jnp.full((L,), win_row[pl.ds(0, L)][0], jnp.int32)
            fpv = jnp.full((L,), win_pos[pl.ds(0, L)][0], jnp.int32)
            for j in range(CHUNK // L):
                win_row[pl.ds(cnt + j * L, L)] = frv
                win_pos[pl.ds(cnt + j * L, L)] = fpv

        nch_w = (cnt + CHUNK - 1) // CHUNK
        ntot = 2 * nch_w

        def fire_gather(b, c):
            t_is_h = c >= nch_w
            ci = jnp.where(t_is_h, c - nch_w, c)
            base = jnp.where(t_is_h, N, 0)

            def ld(j, _):
                d2 = dst2d.at[b]
                s2 = src2d.at[b]
                d2[pl.ds(j * L, L)] = (
                    win_row[pl.ds(ci * CHUNK + j * L, L)] + base)
                s2[pl.ds(j * L, L)] = win_pos[pl.ds(ci * CHUNK + j * L, L)]
                return 0
            lax.fori_loop(0, CHUNK // L, ld, 0)

            @pl.when(jnp.logical_not(t_is_h))
            def _():
                pltpu.make_async_copy(
                    vc_h.at[src2d.at[b]], stage.at[b], sems_g[b]).start()

            @pl.when(t_is_h)
            def _():
                pltpu.make_async_copy(
                    vh_h.at[src2d.at[b]], stage.at[b], sems_g[b]).start()

        def drain(sem, b):
            pltpu.make_async_copy(
                vc_h.at[pl.ds(0, CHUNK)], stage.at[b], sem).wait()

        for b in range(NB):
            @pl.when(b < ntot)
            def _(b=b):
                fire_gather(b, jnp.int32(b))

        def ring(g2, _):
            for b in range(NB):
                c = g2 * NB + b

                @pl.when(c < ntot)
                def _(b=b, c=c):
                    drain(sems_g[b], b)
                    pltpu.make_async_copy(
                        stage.at[b], out_h.at[dst2d.at[b]], sems_s[b]).start()
            for b in range(NB):
                c2 = (g2 + 1) * NB + b

                @pl.when(c2 < ntot)
                def _(b=b, c2=c2):
                    drain(sems_s[b], b)
                    fire_gather(b, c2)
            return 0
        ngroups = (ntot + NB - 1) // NB
        lax.fori_loop(0, ngroups, ring, 0)
        for b in range(NB):
            @pl.when(b < ntot)
            def _(b=b):
                drain(sems_s[b], b)

    k(tmp_all, values_cell, values_hidden, out_ref)


def kernel(cell, hidden, node_idxs, values_cell, values_hidden):
    N, D = cell.shape
    idx = node_idxs.astype(jnp.int32)
    out0 = _tc_copy(cell, hidden)
    tmp_all = _sc_scan(idx, N)
    out_ref = jax.new_ref(out0.reshape(2 * N, D))
    _sc_scatter(out_ref, tmp_all, values_cell, values_hidden, N)
    return out_ref[...].reshape(2, N, D)
